# Initial kernel scaffold; baseline (speedup 1.0000x reference)
import jax
import jax.numpy as jnp
from jax.experimental import pallas as pl


def kernel(sig, sig_lengths):
    raise NotImplementedError("write your pallas kernel here")


# speedup vs baseline: 50.0413x; 50.0413x over previous
import numpy as np
import jax
import jax.numpy as jnp
from jax.experimental import pallas as pl

SR = 16000
NFFT = 512
NMEL = 80
FRAME_LENGTH = 400
FRAME_STEP = 160
FRAME_PER_WIN = 23
FRAME_PER_HOP = 8
NBIN = NFFT // 2 + 1
NBIN_PAD = 384


def _mel_weight_matrix():
    def hz_to_mel(f):
        return 1127.0 * np.log1p(np.asarray(f, dtype=np.float64) / 700.0)
    nyquist = SR / 2.0
    linear_freqs = np.linspace(0.0, nyquist, NBIN)[1:]
    spec_mel = hz_to_mel(linear_freqs)[:, None]
    band_edges = np.linspace(hz_to_mel(80.0), hz_to_mel(7600.0), NMEL + 2)
    lower = band_edges[None, :-2]
    center = band_edges[None, 1:-1]
    upper = band_edges[None, 2:]
    lower_slopes = (spec_mel - lower) / (center - lower)
    upper_slopes = (upper - spec_mel) / (upper - center)
    w = np.maximum(0.0, np.minimum(lower_slopes, upper_slopes))
    return np.pad(w, [[1, 0], [0, 0]]).astype(np.float32)


def _dft_matrices():
    n = np.arange(FRAME_LENGTH, dtype=np.float64)
    k = np.arange(NBIN, dtype=np.float64)
    hann = 0.5 - 0.5 * np.cos(2.0 * np.pi * n / FRAME_LENGTH)
    ang = 2.0 * np.pi * np.outer(n, k) / NFFT
    wre = hann[:, None] * np.cos(ang)
    wim = -hann[:, None] * np.sin(ang)
    w = np.zeros((FRAME_LENGTH, 2 * NBIN_PAD), dtype=np.float32)
    w[:, :NBIN] = wre.astype(np.float32)
    w[:, NBIN_PAD:NBIN_PAD + NBIN] = wim.astype(np.float32)
    melw = np.zeros((NBIN_PAD, NMEL), dtype=np.float32)
    melw[:NBIN] = _mel_weight_matrix()
    return jnp.asarray(w), jnp.asarray(melw)


_W_DFT, _W_MEL = _dft_matrices()


def _stft_mel_win_kernel(nchunk, nframe, nwin, x_ref, w_ref, m_ref, o_ref):
    x = x_ref[0]
    f = jnp.concatenate(
        [x[:nframe], x[1:nframe + 1], x[2:nframe + 2, :FRAME_LENGTH - 2 * FRAME_STEP]],
        axis=1)
    acc = jnp.dot(f, w_ref[...], preferred_element_type=jnp.float32)
    p = acc * acc
    spec = jnp.sqrt(p[:, :NBIN_PAD] + p[:, NBIN_PAD:])
    mel = jnp.dot(spec, m_ref[...], preferred_element_type=jnp.float32)
    lm = jnp.log(mel + 1e-6)
    y = lm[:8 * (nwin + 2)].reshape(nwin + 2, 8, NMEL)
    o_ref[:, 0:8, :] = y[0:nwin]
    o_ref[:, 8:16, :] = y[1:nwin + 1]
    o_ref[:, 16:FRAME_PER_WIN, :] = y[2:nwin + 2, 0:FRAME_PER_WIN - 16]


def kernel(sig, sig_lengths):
    b = sig_lengths.shape[1]
    utt_len = sig.shape[1] // b
    nchunk = utt_len // FRAME_STEP
    nframe = (utt_len - FRAME_LENGTH) // FRAME_STEP + 1
    lastidx = nframe - FRAME_PER_WIN + 1
    nwin = (lastidx + FRAME_PER_HOP - 1) // FRAME_PER_HOP

    chunks = sig.reshape(b, nchunk, FRAME_STEP)
    import functools
    body = functools.partial(_stft_mel_win_kernel, nchunk, nframe, nwin)
    long_batch = pl.pallas_call(
        body,
        grid=(b,),
        in_specs=[
            pl.BlockSpec((1, nchunk, FRAME_STEP), lambda i: (i, 0, 0)),
            pl.BlockSpec((FRAME_LENGTH, 2 * NBIN_PAD), lambda i: (0, 0)),
            pl.BlockSpec((NBIN_PAD, NMEL), lambda i: (0, 0)),
        ],
        out_specs=pl.BlockSpec((nwin, FRAME_PER_WIN, NMEL), lambda i: (i, 0, 0)),
        out_shape=jax.ShapeDtypeStruct((b * nwin, FRAME_PER_WIN, NMEL), jnp.float32),
    )(chunks, _W_DFT, _W_MEL)

    lens = jnp.squeeze(sig_lengths, axis=0)
    mel_lengths = (lens - FRAME_LENGTH) // FRAME_STEP + 1
    lastidx_t = mel_lengths - FRAME_PER_WIN + 1
    sizes_per_mel = ((lastidx_t + FRAME_PER_HOP - 1) // FRAME_PER_HOP).astype(jnp.int32)
    return long_batch, sizes_per_mel
